# baseline (device time: 186391 ns/iter reference)
import jax
import jax.numpy as jnp
from jax import lax
from jax.experimental import pallas as pl
from jax.experimental.pallas import tpu as pltpu

N_DEV = 4
E_LOCAL = 8
E_TOTAL = N_DEV * E_LOCAL
CAP = 160
S = E_TOTAL * CAP
KB = 1024


def _a2a_body(src_ref, dst_ref, send_sems, recv_sems, copy_sem):
    my = lax.axis_index("i")
    local = pltpu.make_async_copy(src_ref.at[my], dst_ref.at[my], copy_sem)
    local.start()
    rdmas = []
    for off in range(1, N_DEV):
        dst = (my + off) % N_DEV
        rdma = pltpu.make_async_remote_copy(
            src_ref=src_ref.at[dst],
            dst_ref=dst_ref.at[my],
            send_sem=send_sems.at[off - 1],
            recv_sem=recv_sems.at[off - 1],
            device_id=(dst,),
            device_id_type=pl.DeviceIdType.MESH,
        )
        rdma.start()
        rdmas.append(rdma)
    local.wait()
    for rdma in rdmas:
        rdma.wait()


def _a2a(x4):
    return pl.pallas_call(
        _a2a_body,
        out_shape=jax.ShapeDtypeStruct(x4.shape, x4.dtype),
        in_specs=[pl.BlockSpec(memory_space=pltpu.HBM)],
        out_specs=pl.BlockSpec(memory_space=pltpu.HBM),
        scratch_shapes=[
            pltpu.SemaphoreType.DMA((N_DEV - 1,)),
            pltpu.SemaphoreType.DMA((N_DEV - 1,)),
            pltpu.SemaphoreType.DMA,
        ],
    )(x4)


def _scatter_mm_body(slot_ref, xs_ref, xg_ref):
    k = pl.program_id(0)
    rows = lax.broadcasted_iota(jnp.int32, (KB, slot_ref.shape[1]), 0) + k * KB
    mask = (rows == slot_ref[...]).astype(jnp.bfloat16)
    xg_ref[...] = jnp.dot(
        mask, xs_ref[...], preferred_element_type=jnp.float32
    ).astype(jnp.bfloat16)


def _expert_mm_body(x_ref, w_ref, y_ref):
    d = x_ref.shape[-1]
    h = y_ref.shape[-1]
    xm = x_ref[:, 0].reshape(N_DEV * CAP, d)
    w = w_ref[0].astype(jnp.bfloat16)
    y = jnp.dot(xm, w, preferred_element_type=jnp.float32)
    y_ref[:, 0] = y.astype(jnp.bfloat16).reshape(N_DEV, CAP, h)


def _gather_mm_body(slot_ref, y_ref, shared_ref, out_ref):
    k = pl.program_id(0)
    cols = lax.broadcasted_iota(jnp.int32, (slot_ref.shape[0], KB), 1) + k * KB
    mask = (cols == slot_ref[...]).astype(jnp.bfloat16)
    part = jnp.dot(mask, y_ref[...], preferred_element_type=jnp.float32)

    @pl.when(k == 0)
    def _():
        out_ref[...] = shared_ref[...] + part

    @pl.when(k > 0)
    def _():
        out_ref[...] += part


def _shared_mm_body(x_ref, w_ref, o_ref):
    o_ref[...] = jnp.dot(x_ref[...], w_ref[...], preferred_element_type=jnp.float32)


def kernel(x, router_W, route_idx, expert_W, shared_W):
    T, D = x.shape
    H = expert_W.shape[-1]

    scores = x @ router_W
    probs = jax.nn.softmax(scores, axis=-1)
    eoh = route_idx == jnp.arange(E_TOTAL, dtype=jnp.int32)[None, :]
    wsel = jnp.sum(jnp.where(eoh, probs, 0.0), axis=1)
    pos = jnp.sum(jnp.where(eoh, jnp.cumsum(eoh.astype(jnp.int32), axis=0) - 1, 0), axis=1)
    e = route_idx[:, 0]
    slot = jnp.where(pos < CAP, e * CAP + pos, -1)
    xs = (x * wsel[:, None]).astype(jnp.bfloat16)

    xg = pl.pallas_call(
        _scatter_mm_body,
        grid=(S // KB,),
        out_shape=jax.ShapeDtypeStruct((S, D), jnp.bfloat16),
        in_specs=[
            pl.BlockSpec((1, T), lambda k: (0, 0)),
            pl.BlockSpec((T, D), lambda k: (0, 0)),
        ],
        out_specs=pl.BlockSpec((KB, D), lambda k: (k, 0)),
    )(slot.reshape(1, T), xs)

    recv = _a2a(xg.reshape(N_DEV, E_LOCAL, CAP, D))

    y = pl.pallas_call(
        _expert_mm_body,
        grid=(E_LOCAL,),
        out_shape=jax.ShapeDtypeStruct((N_DEV, E_LOCAL, CAP, H), jnp.bfloat16),
        in_specs=[
            pl.BlockSpec((N_DEV, 1, CAP, D), lambda j: (0, j, 0, 0)),
            pl.BlockSpec((1, D, H), lambda j: (j, 0, 0)),
        ],
        out_specs=pl.BlockSpec((N_DEV, 1, CAP, H), lambda j: (0, j, 0, 0)),
    )(recv, expert_W)

    y_back = _a2a(y)

    shared_out = pl.pallas_call(
        _shared_mm_body,
        out_shape=jax.ShapeDtypeStruct((T, H), jnp.float32),
        in_specs=[
            pl.BlockSpec(memory_space=pltpu.VMEM),
            pl.BlockSpec(memory_space=pltpu.VMEM),
        ],
        out_specs=pl.BlockSpec(memory_space=pltpu.VMEM),
    )(x, shared_W)

    out = pl.pallas_call(
        _gather_mm_body,
        grid=(S // KB,),
        out_shape=jax.ShapeDtypeStruct((T, H), jnp.float32),
        in_specs=[
            pl.BlockSpec((T, 1), lambda k: (0, 0)),
            pl.BlockSpec((KB, H), lambda k: (k, 0)),
            pl.BlockSpec((T, H), lambda k: (0, 0)),
        ],
        out_specs=pl.BlockSpec((T, H), lambda k: (0, 0)),
    )(slot.reshape(T, 1), y_back.reshape(S, H), shared_out)

    return out


# device time: 178949 ns/iter; 1.0416x vs baseline; 1.0416x over previous
import jax
import jax.numpy as jnp
from jax import lax
from jax.experimental import pallas as pl
from jax.experimental.pallas import tpu as pltpu

N_DEV = 4
E_LOCAL = 8
E_TOTAL = N_DEV * E_LOCAL
CAP = 160
BLK = E_LOCAL * CAP
S = N_DEV * BLK
KB = 1024

USE_FUSED_K1 = True
USE_FUSED_K2 = True


def _a2a_body(src_ref, dst_ref, send_sems, recv_sems, copy_sem):
    my = lax.axis_index("i")
    local = pltpu.make_async_copy(src_ref.at[my], dst_ref.at[my], copy_sem)
    local.start()
    rdmas = []
    for off in range(1, N_DEV):
        dst = (my + off) % N_DEV
        rdma = pltpu.make_async_remote_copy(
            src_ref=src_ref.at[dst],
            dst_ref=dst_ref.at[my],
            send_sem=send_sems.at[off - 1],
            recv_sem=recv_sems.at[off - 1],
            device_id=(dst,),
            device_id_type=pl.DeviceIdType.MESH,
        )
        rdma.start()
        rdmas.append(rdma)
    local.wait()
    for rdma in rdmas:
        rdma.wait()


def _a2a(x4):
    return pl.pallas_call(
        _a2a_body,
        out_shape=jax.ShapeDtypeStruct(x4.shape, x4.dtype),
        in_specs=[pl.BlockSpec(memory_space=pltpu.HBM)],
        out_specs=pl.BlockSpec(memory_space=pltpu.HBM),
        scratch_shapes=[
            pltpu.SemaphoreType.DMA((N_DEV - 1,)),
            pltpu.SemaphoreType.DMA((N_DEV - 1,)),
            pltpu.SemaphoreType.DMA,
        ],
    )(x4)


def _scatter_mm_body(slot_ref, xs_ref, xg_ref):
    k = pl.program_id(0)
    rows = lax.broadcasted_iota(jnp.int32, (KB, slot_ref.shape[1]), 0) + k * KB
    mask = (rows == slot_ref[...]).astype(jnp.bfloat16)
    xg_ref[...] = jnp.dot(
        mask, xs_ref[...], preferred_element_type=jnp.float32
    ).astype(jnp.bfloat16)


def _expert_mm_body(x_ref, w_ref, y_ref):
    d = x_ref.shape[-1]
    h = y_ref.shape[-1]
    xm = x_ref[:, 0].reshape(N_DEV * CAP, d)
    w = w_ref[0].astype(jnp.bfloat16)
    y = jnp.dot(xm, w, preferred_element_type=jnp.float32)
    y_ref[:, 0] = y.astype(jnp.bfloat16).reshape(N_DEV, CAP, h)


def _gather_mm_body(slot_ref, y_ref, shared_ref, out_ref):
    k = pl.program_id(0)
    cols = lax.broadcasted_iota(jnp.int32, (slot_ref.shape[0], KB), 1) + k * KB
    mask = (cols == slot_ref[...]).astype(jnp.bfloat16)
    part = jnp.dot(mask, y_ref[...], preferred_element_type=jnp.float32)

    @pl.when(k == 0)
    def _():
        out_ref[...] = shared_ref[...] + part

    @pl.when(k > 0)
    def _():
        out_ref[...] += part


def _shared_mm_body(x_ref, w_ref, o_ref):
    o_ref[...] = jnp.dot(x_ref[...], w_ref[...], preferred_element_type=jnp.float32)


def _group_dispatch_body(slot_ref, xs_ref, recv_ref, tiles, send_sems, recv_sems, copy_sem):
    my = lax.axis_index("i")
    T = slot_ref.shape[1]
    d = xs_ref.shape[-1]

    barrier = pltpu.get_barrier_semaphore()
    for off in range(1, N_DEV):
        pl.semaphore_signal(
            barrier, 1,
            device_id=((my + off) % N_DEV,),
            device_id_type=pl.DeviceIdType.MESH,
        )
    pl.semaphore_wait(barrier, N_DEV - 1)

    def bucket_block(dst):
        rows = lax.broadcasted_iota(jnp.int32, (BLK, T), 0) + dst * BLK
        mask = (rows == slot_ref[...]).astype(jnp.bfloat16)
        xg = jnp.dot(mask, xs_ref[...], preferred_element_type=jnp.float32)
        return xg.astype(jnp.bfloat16).reshape(E_LOCAL, CAP, d)

    descs = []
    for k in range(N_DEV - 1):
        dst = (my + 1 + k) % N_DEV
        if k >= 2:
            descs[k - 2].wait_send()
        tiles[k % 2] = bucket_block(dst)
        rdma = pltpu.make_async_remote_copy(
            src_ref=tiles.at[k % 2],
            dst_ref=recv_ref.at[my],
            send_sem=send_sems.at[k],
            recv_sem=recv_sems.at[k],
            device_id=(dst,),
            device_id_type=pl.DeviceIdType.MESH,
        )
        rdma.start()
        descs.append(rdma)
    descs[1].wait_send()
    tiles[1] = bucket_block(my)
    own = pltpu.make_async_copy(tiles.at[1], recv_ref.at[my], copy_sem)
    own.start()
    own.wait()
    descs[2].wait_send()
    for rdma in descs:
        rdma.wait_recv()


def _expert_combine_body(
    recv_ref, w_ref, slot_ref, x_ref, sw_ref, out_ref, tiles, yb, send_sems, recv_sems
):
    my = lax.axis_index("i")
    T, h = out_ref.shape

    barrier = pltpu.get_barrier_semaphore()
    for off in range(1, N_DEV):
        pl.semaphore_signal(
            barrier, 1,
            device_id=((my + off) % N_DEV,),
            device_id_type=pl.DeviceIdType.MESH,
        )
    pl.semaphore_wait(barrier, N_DEV - 1)

    descs = []
    idx = 0
    for k in range(N_DEV - 1):
        s = (my + 1 + k) % N_DEV
        for j in range(E_LOCAL):
            if idx >= 2:
                descs[idx - 2].wait_send()
            xm = recv_ref[s, j]
            tiles[idx % 2] = jnp.dot(
                xm, w_ref[j], preferred_element_type=jnp.float32
            ).astype(jnp.bfloat16)
            rdma = pltpu.make_async_remote_copy(
                src_ref=tiles.at[idx % 2],
                dst_ref=yb.at[my, j],
                send_sem=send_sems.at[k, j],
                recv_sem=recv_sems.at[k, j],
                device_id=(s,),
                device_id_type=pl.DeviceIdType.MESH,
            )
            rdma.start()
            descs.append(rdma)
            idx += 1
    for j in range(E_LOCAL):
        yb[my, j] = jnp.dot(
            recv_ref[my, j], w_ref[j], preferred_element_type=jnp.float32
        ).astype(jnp.bfloat16)

    out_ref[...] = jnp.dot(x_ref[...], sw_ref[...], preferred_element_type=jnp.float32)

    for kk in range(N_DEV):
        p = (my + kk) % N_DEV
        if kk > 0:
            for j in range(E_LOCAL):
                pltpu.make_async_remote_copy(
                    src_ref=tiles.at[0],
                    dst_ref=yb.at[p, j],
                    send_sem=send_sems.at[0, 0],
                    recv_sem=recv_sems.at[3 - kk, j],
                    device_id=(p,),
                    device_id_type=pl.DeviceIdType.MESH,
                ).wait_recv()
        cols = lax.broadcasted_iota(jnp.int32, (T, BLK), 1) + p * BLK
        mask = (cols == slot_ref[...]).astype(jnp.bfloat16)
        ysrc = yb[p].reshape(BLK, h)
        out_ref[...] += jnp.dot(mask, ysrc, preferred_element_type=jnp.float32)

    for rdma in descs[-2:]:
        rdma.wait_send()


def kernel(x, router_W, route_idx, expert_W, shared_W):
    T, D = x.shape
    H = expert_W.shape[-1]

    scores = x @ router_W
    probs = jax.nn.softmax(scores, axis=-1)
    eoh = route_idx == jnp.arange(E_TOTAL, dtype=jnp.int32)[None, :]
    wsel = jnp.sum(jnp.where(eoh, probs, 0.0), axis=1)
    pos = jnp.sum(jnp.where(eoh, jnp.cumsum(eoh.astype(jnp.int32), axis=0) - 1, 0), axis=1)
    e = route_idx[:, 0]
    slot = jnp.where(pos < CAP, e * CAP + pos, -1)
    xs = (x * wsel[:, None]).astype(jnp.bfloat16)

    if USE_FUSED_K1:
        recv = pl.pallas_call(
            _group_dispatch_body,
            out_shape=jax.ShapeDtypeStruct((N_DEV, E_LOCAL, CAP, D), jnp.bfloat16),
            in_specs=[
                pl.BlockSpec(memory_space=pltpu.VMEM),
                pl.BlockSpec(memory_space=pltpu.VMEM),
            ],
            out_specs=pl.BlockSpec(memory_space=pltpu.HBM),
            scratch_shapes=[
                pltpu.VMEM((2, E_LOCAL, CAP, D), jnp.bfloat16),
                pltpu.SemaphoreType.DMA((N_DEV - 1,)),
                pltpu.SemaphoreType.DMA((N_DEV - 1,)),
                pltpu.SemaphoreType.DMA,
            ],
            compiler_params=pltpu.CompilerParams(collective_id=0),
        )(slot.reshape(1, T), xs)
    else:
        xg = pl.pallas_call(
            _scatter_mm_body,
            grid=(S // KB,),
            out_shape=jax.ShapeDtypeStruct((S, D), jnp.bfloat16),
            in_specs=[
                pl.BlockSpec((1, T), lambda k: (0, 0)),
                pl.BlockSpec((T, D), lambda k: (0, 0)),
            ],
            out_specs=pl.BlockSpec((KB, D), lambda k: (k, 0)),
        )(slot.reshape(1, T), xs)
        recv = _a2a(xg.reshape(N_DEV, E_LOCAL, CAP, D))

    if USE_FUSED_K2:
        out = pl.pallas_call(
            _expert_combine_body,
            out_shape=jax.ShapeDtypeStruct((T, H), jnp.float32),
            in_specs=[
                pl.BlockSpec(memory_space=pltpu.VMEM),
                pl.BlockSpec(memory_space=pltpu.VMEM),
                pl.BlockSpec(memory_space=pltpu.VMEM),
                pl.BlockSpec(memory_space=pltpu.VMEM),
                pl.BlockSpec(memory_space=pltpu.VMEM),
            ],
            out_specs=pl.BlockSpec(memory_space=pltpu.VMEM),
            scratch_shapes=[
                pltpu.VMEM((2, CAP, H), jnp.bfloat16),
                pltpu.VMEM((N_DEV, E_LOCAL, CAP, H), jnp.bfloat16),
                pltpu.SemaphoreType.DMA((N_DEV - 1, E_LOCAL)),
                pltpu.SemaphoreType.DMA((N_DEV - 1, E_LOCAL)),
            ],
            compiler_params=pltpu.CompilerParams(collective_id=1),
        )(recv, expert_W.astype(jnp.bfloat16), slot.reshape(T, 1), x, shared_W)
        return out

    y = pl.pallas_call(
        _expert_mm_body,
        grid=(E_LOCAL,),
        out_shape=jax.ShapeDtypeStruct((N_DEV, E_LOCAL, CAP, H), jnp.bfloat16),
        in_specs=[
            pl.BlockSpec((N_DEV, 1, CAP, D), lambda j: (0, j, 0, 0)),
            pl.BlockSpec((1, D, H), lambda j: (j, 0, 0)),
        ],
        out_specs=pl.BlockSpec((N_DEV, 1, CAP, H), lambda j: (0, j, 0, 0)),
    )(recv, expert_W)

    y_back = _a2a(y)

    shared_out = pl.pallas_call(
        _shared_mm_body,
        out_shape=jax.ShapeDtypeStruct((T, H), jnp.float32),
        in_specs=[
            pl.BlockSpec(memory_space=pltpu.VMEM),
            pl.BlockSpec(memory_space=pltpu.VMEM),
        ],
        out_specs=pl.BlockSpec(memory_space=pltpu.VMEM),
    )(x, shared_W)

    out = pl.pallas_call(
        _gather_mm_body,
        grid=(S // KB,),
        out_shape=jax.ShapeDtypeStruct((T, H), jnp.float32),
        in_specs=[
            pl.BlockSpec((T, 1), lambda k: (0, 0)),
            pl.BlockSpec((KB, H), lambda k: (k, 0)),
            pl.BlockSpec((T, H), lambda k: (0, 0)),
        ],
        out_specs=pl.BlockSpec((T, H), lambda k: (0, 0)),
    )(slot.reshape(T, 1), y_back.reshape(S, H), shared_out)

    return out


# device time: 178882 ns/iter; 1.0420x vs baseline; 1.0004x over previous
import jax
import jax.numpy as jnp
from jax import lax
from jax.experimental import pallas as pl
from jax.experimental.pallas import tpu as pltpu

N_DEV = 4
E_LOCAL = 8
E_TOTAL = N_DEV * E_LOCAL
CAP = 160
BLK = E_LOCAL * CAP
S = N_DEV * BLK
KB = 1024

USE_FUSED_K1 = True
USE_FUSED_K2 = True


def _a2a_body(src_ref, dst_ref, send_sems, recv_sems, copy_sem):
    my = lax.axis_index("i")
    local = pltpu.make_async_copy(src_ref.at[my], dst_ref.at[my], copy_sem)
    local.start()
    rdmas = []
    for off in range(1, N_DEV):
        dst = (my + off) % N_DEV
        rdma = pltpu.make_async_remote_copy(
            src_ref=src_ref.at[dst],
            dst_ref=dst_ref.at[my],
            send_sem=send_sems.at[off - 1],
            recv_sem=recv_sems.at[off - 1],
            device_id=(dst,),
            device_id_type=pl.DeviceIdType.MESH,
        )
        rdma.start()
        rdmas.append(rdma)
    local.wait()
    for rdma in rdmas:
        rdma.wait()


def _a2a(x4):
    return pl.pallas_call(
        _a2a_body,
        out_shape=jax.ShapeDtypeStruct(x4.shape, x4.dtype),
        in_specs=[pl.BlockSpec(memory_space=pltpu.HBM)],
        out_specs=pl.BlockSpec(memory_space=pltpu.HBM),
        scratch_shapes=[
            pltpu.SemaphoreType.DMA((N_DEV - 1,)),
            pltpu.SemaphoreType.DMA((N_DEV - 1,)),
            pltpu.SemaphoreType.DMA,
        ],
    )(x4)


def _scatter_mm_body(slot_ref, xs_ref, xg_ref):
    k = pl.program_id(0)
    rows = lax.broadcasted_iota(jnp.int32, (KB, slot_ref.shape[1]), 0) + k * KB
    mask = (rows == slot_ref[...]).astype(jnp.bfloat16)
    xg_ref[...] = jnp.dot(
        mask, xs_ref[...], preferred_element_type=jnp.float32
    ).astype(jnp.bfloat16)


def _expert_mm_body(x_ref, w_ref, y_ref):
    d = x_ref.shape[-1]
    h = y_ref.shape[-1]
    xm = x_ref[:, 0].reshape(N_DEV * CAP, d)
    w = w_ref[0].astype(jnp.bfloat16)
    y = jnp.dot(xm, w, preferred_element_type=jnp.float32)
    y_ref[:, 0] = y.astype(jnp.bfloat16).reshape(N_DEV, CAP, h)


def _gather_mm_body(slot_ref, y_ref, shared_ref, out_ref):
    k = pl.program_id(0)
    cols = lax.broadcasted_iota(jnp.int32, (slot_ref.shape[0], KB), 1) + k * KB
    mask = (cols == slot_ref[...]).astype(jnp.bfloat16)
    part = jnp.dot(mask, y_ref[...], preferred_element_type=jnp.float32)

    @pl.when(k == 0)
    def _():
        out_ref[...] = shared_ref[...] + part

    @pl.when(k > 0)
    def _():
        out_ref[...] += part


def _shared_mm_body(x_ref, w_ref, o_ref):
    o_ref[...] = jnp.dot(x_ref[...], w_ref[...], preferred_element_type=jnp.float32)


def _group_dispatch_body(slot_ref, xs_ref, recv_ref, tiles, send_sems, recv_sems, copy_sem):
    my = lax.axis_index("i")
    T = slot_ref.shape[1]
    d = xs_ref.shape[-1]

    barrier = pltpu.get_barrier_semaphore()
    for off in range(1, N_DEV):
        pl.semaphore_signal(
            barrier, 1,
            device_id=((my + off) % N_DEV,),
            device_id_type=pl.DeviceIdType.MESH,
        )
    pl.semaphore_wait(barrier, N_DEV - 1)

    def bucket_block(dst):
        rows = lax.broadcasted_iota(jnp.int32, (BLK, T), 0) + dst * BLK
        mask = (rows == slot_ref[...]).astype(jnp.bfloat16)
        xg = jnp.dot(mask, xs_ref[...], preferred_element_type=jnp.float32)
        return xg.astype(jnp.bfloat16).reshape(E_LOCAL, CAP, d)

    descs = []
    for k in range(N_DEV - 1):
        dst = (my + 1 + k) % N_DEV
        if k >= 2:
            descs[k - 2].wait_send()
        tiles[k % 2] = bucket_block(dst)
        rdma = pltpu.make_async_remote_copy(
            src_ref=tiles.at[k % 2],
            dst_ref=recv_ref.at[my],
            send_sem=send_sems.at[k],
            recv_sem=recv_sems.at[k],
            device_id=(dst,),
            device_id_type=pl.DeviceIdType.MESH,
        )
        rdma.start()
        descs.append(rdma)
    descs[1].wait_send()
    tiles[1] = bucket_block(my)
    own = pltpu.make_async_copy(tiles.at[1], recv_ref.at[my], copy_sem)
    own.start()
    own.wait()
    descs[2].wait_send()
    for rdma in descs:
        rdma.wait_recv()


def _expert_combine_body(
    recv_ref, w_ref, slot_ref, x_ref, sw_ref, out_ref, tiles, yb, send_sems, recv_sems
):
    my = lax.axis_index("i")
    T, h = out_ref.shape

    barrier = pltpu.get_barrier_semaphore()
    for off in range(1, N_DEV):
        pl.semaphore_signal(
            barrier, 1,
            device_id=((my + off) % N_DEV,),
            device_id_type=pl.DeviceIdType.MESH,
        )
    pl.semaphore_wait(barrier, N_DEV - 1)

    descs = []
    idx = 0
    for k in range(N_DEV - 1):
        s = (my + 1 + k) % N_DEV
        for j in range(E_LOCAL):
            if idx >= 2:
                descs[idx - 2].wait_send()
            xm = recv_ref[s, j]
            tiles[idx % 2] = jnp.dot(
                xm, w_ref[j], preferred_element_type=jnp.float32
            ).astype(jnp.bfloat16)
            rdma = pltpu.make_async_remote_copy(
                src_ref=tiles.at[idx % 2],
                dst_ref=yb.at[my, j],
                send_sem=send_sems.at[k, j],
                recv_sem=recv_sems.at[k, j],
                device_id=(s,),
                device_id_type=pl.DeviceIdType.MESH,
            )
            rdma.start()
            descs.append(rdma)
            idx += 1
    for j in range(E_LOCAL):
        yb[my, j] = jnp.dot(
            recv_ref[my, j], w_ref[j], preferred_element_type=jnp.float32
        ).astype(jnp.bfloat16)

    out_ref[...] = jnp.dot(x_ref[...], sw_ref[...], preferred_element_type=jnp.float32)

    for kk in (0, 3, 2, 1):
        p = (my + kk) % N_DEV
        if kk > 0:
            for j in range(E_LOCAL):
                pltpu.make_async_remote_copy(
                    src_ref=tiles.at[0],
                    dst_ref=yb.at[p, j],
                    send_sem=send_sems.at[0, 0],
                    recv_sem=recv_sems.at[3 - kk, j],
                    device_id=(p,),
                    device_id_type=pl.DeviceIdType.MESH,
                ).wait_recv()
        cols = lax.broadcasted_iota(jnp.int32, (T, BLK), 1) + p * BLK
        mask = (cols == slot_ref[...]).astype(jnp.bfloat16)
        ysrc = yb[p].reshape(BLK, h)
        out_ref[...] += jnp.dot(mask, ysrc, preferred_element_type=jnp.float32)

    for rdma in descs[-2:]:
        rdma.wait_send()


def kernel(x, router_W, route_idx, expert_W, shared_W):
    T, D = x.shape
    H = expert_W.shape[-1]

    scores = x @ router_W
    probs = jax.nn.softmax(scores, axis=-1)
    eoh = route_idx == jnp.arange(E_TOTAL, dtype=jnp.int32)[None, :]
    wsel = jnp.sum(jnp.where(eoh, probs, 0.0), axis=1)
    pos = jnp.sum(jnp.where(eoh, jnp.cumsum(eoh.astype(jnp.int32), axis=0) - 1, 0), axis=1)
    e = route_idx[:, 0]
    slot = jnp.where(pos < CAP, e * CAP + pos, -1)
    xs = (x * wsel[:, None]).astype(jnp.bfloat16)

    if USE_FUSED_K1:
        recv = pl.pallas_call(
            _group_dispatch_body,
            out_shape=jax.ShapeDtypeStruct((N_DEV, E_LOCAL, CAP, D), jnp.bfloat16),
            in_specs=[
                pl.BlockSpec(memory_space=pltpu.VMEM),
                pl.BlockSpec(memory_space=pltpu.VMEM),
            ],
            out_specs=pl.BlockSpec(memory_space=pltpu.HBM),
            scratch_shapes=[
                pltpu.VMEM((2, E_LOCAL, CAP, D), jnp.bfloat16),
                pltpu.SemaphoreType.DMA((N_DEV - 1,)),
                pltpu.SemaphoreType.DMA((N_DEV - 1,)),
                pltpu.SemaphoreType.DMA,
            ],
            compiler_params=pltpu.CompilerParams(collective_id=0),
        )(slot.reshape(1, T), xs)
    else:
        xg = pl.pallas_call(
            _scatter_mm_body,
            grid=(S // KB,),
            out_shape=jax.ShapeDtypeStruct((S, D), jnp.bfloat16),
            in_specs=[
                pl.BlockSpec((1, T), lambda k: (0, 0)),
                pl.BlockSpec((T, D), lambda k: (0, 0)),
            ],
            out_specs=pl.BlockSpec((KB, D), lambda k: (k, 0)),
        )(slot.reshape(1, T), xs)
        recv = _a2a(xg.reshape(N_DEV, E_LOCAL, CAP, D))

    if USE_FUSED_K2:
        out = pl.pallas_call(
            _expert_combine_body,
            out_shape=jax.ShapeDtypeStruct((T, H), jnp.float32),
            in_specs=[
                pl.BlockSpec(memory_space=pltpu.VMEM),
                pl.BlockSpec(memory_space=pltpu.VMEM),
                pl.BlockSpec(memory_space=pltpu.VMEM),
                pl.BlockSpec(memory_space=pltpu.VMEM),
                pl.BlockSpec(memory_space=pltpu.VMEM),
            ],
            out_specs=pl.BlockSpec(memory_space=pltpu.VMEM),
            scratch_shapes=[
                pltpu.VMEM((2, CAP, H), jnp.bfloat16),
                pltpu.VMEM((N_DEV, E_LOCAL, CAP, H), jnp.bfloat16),
                pltpu.SemaphoreType.DMA((N_DEV - 1, E_LOCAL)),
                pltpu.SemaphoreType.DMA((N_DEV - 1, E_LOCAL)),
            ],
            compiler_params=pltpu.CompilerParams(collective_id=1),
        )(recv, expert_W.astype(jnp.bfloat16), slot.reshape(T, 1), x, shared_W)
        return out

    y = pl.pallas_call(
        _expert_mm_body,
        grid=(E_LOCAL,),
        out_shape=jax.ShapeDtypeStruct((N_DEV, E_LOCAL, CAP, H), jnp.bfloat16),
        in_specs=[
            pl.BlockSpec((N_DEV, 1, CAP, D), lambda j: (0, j, 0, 0)),
            pl.BlockSpec((1, D, H), lambda j: (j, 0, 0)),
        ],
        out_specs=pl.BlockSpec((N_DEV, 1, CAP, H), lambda j: (0, j, 0, 0)),
    )(recv, expert_W)

    y_back = _a2a(y)

    shared_out = pl.pallas_call(
        _shared_mm_body,
        out_shape=jax.ShapeDtypeStruct((T, H), jnp.float32),
        in_specs=[
            pl.BlockSpec(memory_space=pltpu.VMEM),
            pl.BlockSpec(memory_space=pltpu.VMEM),
        ],
        out_specs=pl.BlockSpec(memory_space=pltpu.VMEM),
    )(x, shared_W)

    out = pl.pallas_call(
        _gather_mm_body,
        grid=(S // KB,),
        out_shape=jax.ShapeDtypeStruct((T, H), jnp.float32),
        in_specs=[
            pl.BlockSpec((T, 1), lambda k: (0, 0)),
            pl.BlockSpec((KB, H), lambda k: (k, 0)),
            pl.BlockSpec((T, H), lambda k: (0, 0)),
        ],
        out_specs=pl.BlockSpec((T, H), lambda k: (0, 0)),
    )(slot.reshape(T, 1), y_back.reshape(S, H), shared_out)

    return out
